# baseline (speedup 1.0000x reference)
import jax
import jax.numpy as jnp
from jax.experimental import pallas as pl
from jax.experimental.pallas import tpu as pltpu

_VOCAB = 50257
_SEQ = 2048
_DIM = 768
_BATCH = 8

_TOK_BLK = 256
_N_TOK = _BATCH * _SEQ
_N_BLK = _N_TOK // _TOK_BLK
_CORES = 2
_BLK_PER_CORE = _N_BLK // _CORES


def _emb_kernel(tok_ref, wT_hbm, pos_ref, out_ref, scr_ref, sem):
    c = pl.program_id(0)
    j = pl.program_id(1)
    base = (c * _BLK_PER_CORE + j) * _TOK_BLK
    for mi in range(_TOK_BLK):
        t = tok_ref[base + mi]
        pltpu.make_async_copy(wT_hbm.at[t], scr_ref.at[mi], sem).start()
    pltpu.make_async_copy(
        wT_hbm.at[pl.ds(0, _TOK_BLK)], scr_ref, sem
    ).wait()
    pos_start = base % _SEQ
    out_ref[...] = scr_ref[...] + pos_ref[pl.ds(pos_start, _TOK_BLK)]


def kernel(tokens, W_emb, W_pos):
    wT = W_emb.T.reshape(_VOCAB, 1, _DIM)
    tok = tokens.reshape(_N_TOK)
    pos3 = W_pos.reshape(_SEQ, 1, _DIM)

    out = pl.pallas_call(
        _emb_kernel,
        out_shape=jax.ShapeDtypeStruct((_N_TOK, 1, _DIM), jnp.float32),
        grid_spec=pltpu.PrefetchScalarGridSpec(
            num_scalar_prefetch=1,
            grid=(_CORES, _BLK_PER_CORE),
            in_specs=[
                pl.BlockSpec(memory_space=pl.ANY),
                pl.BlockSpec((_SEQ, 1, _DIM), lambda c, j, tok_ref: (0, 0, 0)),
            ],
            out_specs=pl.BlockSpec(
                (_TOK_BLK, 1, _DIM),
                lambda c, j, tok_ref: (c * _BLK_PER_CORE + j, 0, 0),
            ),
            scratch_shapes=[
                pltpu.VMEM((_TOK_BLK, 1, _DIM), jnp.float32),
                pltpu.SemaphoreType.DMA,
            ],
        ),
        compiler_params=pltpu.CompilerParams(
            dimension_semantics=("parallel", "arbitrary"),
        ),
        name="embedding_gather",
    )(tok, wT, pos3)
    return out.reshape(_BATCH, _SEQ, _DIM)
